# baseline (speedup 1.0000x reference)
import functools
import math

import jax
import jax.numpy as jnp
from jax import lax
from jax.experimental import pallas as pl
from jax.experimental.pallas import tpu as pltpu
from jax.experimental.pallas import tpu_sc as plsc

_H = 4
_K = 16
_LANES = 16


def _k1_body(x_ref, sum_ref, max_ref):
    i = pl.program_id(0)
    x = x_ref[...]
    s = jnp.sum(x, axis=1)
    m = jnp.max(x, axis=1)

    @pl.when(i == 0)
    def _():
        sum_ref[...] = s
        max_ref[...] = m

    @pl.when(i > 0)
    def _():
        sum_ref[...] = sum_ref[...] + s
        max_ref[...] = jnp.maximum(max_ref[...], m)


def _reduce_x(X, blk_n):
    b, n, d = X.shape
    grid = n // blk_n
    return pl.pallas_call(
        _k1_body,
        grid=(grid,),
        in_specs=[pl.BlockSpec((b, blk_n, d), lambda i: (0, i, 0))],
        out_specs=[
            pl.BlockSpec((b, d), lambda i: (0, 0)),
            pl.BlockSpec((b, d), lambda i: (0, 0)),
        ],
        out_shape=[
            jax.ShapeDtypeStruct((b, d), jnp.float32),
            jax.ShapeDtypeStruct((b, d), jnp.float32),
        ],
    )(X)


def _k2_body(inv_n, e_per_blk, sumx_ref, maxx_ref, wc_ref, add_ref, proto_ref):
    d = sumx_ref.shape[1]
    avg = sumx_ref[...] * inv_n
    mx = maxx_ref[...]
    for e in range(e_per_blk):
        sl = pl.ds(e * d, d)
        off = jnp.dot(avg, wc_ref[0:d, sl], preferred_element_type=jnp.float32)
        off = off + jnp.dot(mx, wc_ref[d : 2 * d, sl],
                            preferred_element_type=jnp.float32)
        proto_ref[:, 0, e, :] = off + add_ref[0, e][None, :]


def _prototypes(sum_X, max_X, Wc, base_plus_bc, n_tokens, e_per_blk):
    b, d = sum_X.shape
    total = base_plus_bc.shape[0]
    grid = total // e_per_blk
    body = functools.partial(_k2_body, 1.0 / n_tokens, e_per_blk)
    return pl.pallas_call(
        body,
        grid=(grid,),
        in_specs=[
            pl.BlockSpec((b, d), lambda i: (0, 0)),
            pl.BlockSpec((b, d), lambda i: (0, 0)),
            pl.BlockSpec((2 * d, e_per_blk * d), lambda i: (0, i)),
            pl.BlockSpec((1, e_per_blk, d), lambda i: (i, 0, 0)),
        ],
        out_specs=pl.BlockSpec((b, 1, e_per_blk, d), lambda i: (0, i, 0, 0)),
        out_shape=jax.ShapeDtypeStruct((b, grid, e_per_blk, d), jnp.float32),
    )(sum_X, max_X, Wc, base_plus_bc.reshape(grid, e_per_blk, d)
      ).reshape(b, total, d)


def _k3_body(scale, n_tokens, proto_ref, wp_ref, bp_ref, sumx_ref,
             w2_ref, cb_ref, gs_ref):
    b = proto_ref.shape[0]
    for bi in range(b):
        p = proto_ref[bi]
        w2 = lax.dot_general(wp_ref[...], p, (((1,), (1,)), ((), ())),
                             preferred_element_type=jnp.float32) * scale
        w2_ref[bi] = w2
        cb = lax.dot_general(bp_ref[...], p, (((1,), (1,)), ((), ())),
                             preferred_element_type=jnp.float32) * scale
        sx = sumx_ref[pl.ds(bi, 1), :]
        gs = jnp.dot(sx, w2, preferred_element_type=jnp.float32) \
            + n_tokens * cb
        cb_ref[pl.ds(bi, 1), :] = cb
        gs_ref[pl.ds(bi, 1), :] = gs


def _edge_weights(protos, Wp, bp, sum_X, n_tokens):
    b, total, d = protos.shape
    scale = 1.0 / (_H * math.sqrt(d / _H))
    body = functools.partial(_k3_body, scale, float(n_tokens))
    return pl.pallas_call(
        body,
        out_shape=[
            jax.ShapeDtypeStruct((b, d, total), jnp.float32),
            jax.ShapeDtypeStruct((b, total), jnp.float32),
            jax.ShapeDtypeStruct((b, total), jnp.float32),
        ],
    )(protos, Wp, bp.reshape(1, d), sum_X)


def _sc_mask_body(batches, total, gs_hbm, out_hbm, row_v, mask_v):
    cid = lax.axis_index("c")
    sid = lax.axis_index("s")
    wid = sid * 2 + cid
    nchunk = total // _LANES

    @pl.when(wid < batches)
    def _():
        base = wid * total
        pltpu.sync_copy(gs_hbm.at[pl.ds(base, total)],
                        row_v.at[pl.ds(0, total)])
        chunks = [row_v[pl.ds(i * _LANES, _LANES)] for i in range(nchunk)]
        lane = lax.iota(jnp.int32, _LANES)
        eidx = [lane + i * _LANES for i in range(nchunk)]
        one = jnp.ones((_LANES,), jnp.int32)
        zero = jnp.zeros((_LANES,), jnp.int32)

        def body(j, accs):
            s = row_v[pl.ds(j, _LANES)][0]
            sj = jnp.full((_LANES,), s, jnp.float32)
            out = []
            for i in range(nchunk):
                gt = sj > chunks[i]
                eq = (sj == chunks[i]) & (j < eidx[i])
                out.append(accs[i] + jnp.where(gt, one, zero)
                           + jnp.where(eq, one, zero))
            return tuple(out)

        accs = lax.fori_loop(0, total, body, tuple([zero] * nchunk))
        fone = jnp.ones((_LANES,), jnp.float32)
        fzero = jnp.zeros((_LANES,), jnp.float32)
        for i in range(nchunk):
            mask_v[pl.ds(i * _LANES, _LANES)] = jnp.where(accs[i] < _K,
                                                          fone, fzero)
        pltpu.sync_copy(mask_v, out_hbm.at[pl.ds(base, total)])


def _topk_mask_sc(gs):
    b, total = gs.shape
    mesh = plsc.VectorSubcoreMesh(core_axis_name="c", subcore_axis_name="s")
    body = functools.partial(_sc_mask_body, b, total)
    kern = pl.kernel(
        body,
        out_type=jax.ShapeDtypeStruct((b * total,), jnp.float32),
        mesh=mesh,
        scratch_types=[
            pltpu.VMEM((total + _LANES,), jnp.float32),
            pltpu.VMEM((total,), jnp.float32),
        ],
    )
    return kern(gs.reshape(-1)).reshape(b, total)


def _k4_body(x_ref, w2_ref, cb_ref, a_ref, m_out, s_out, m_acc, s_acc):
    i = pl.program_id(0)
    nb = pl.num_programs(0)
    b = x_ref.shape[0]
    for bi in range(b):
        a = jnp.dot(x_ref[bi], w2_ref[bi], preferred_element_type=jnp.float32)
        a = a + cb_ref[bi][None, :]
        a_ref[bi] = a
        m_blk = jnp.max(a, axis=0)

        @pl.when(i == 0)
        def _():
            m_acc[bi] = m_blk
            s_acc[bi] = jnp.sum(jnp.exp(a - m_blk[None, :]), axis=0)

        @pl.when(i > 0)
        def _():
            m_old = m_acc[bi]
            m_new = jnp.maximum(m_old, m_blk)
            s_acc[bi] = s_acc[bi] * jnp.exp(m_old - m_new) + \
                jnp.sum(jnp.exp(a - m_new[None, :]), axis=0)
            m_acc[bi] = m_new

    @pl.when(i == nb - 1)
    def _():
        m_out[...] = m_acc[...]
        s_out[...] = s_acc[...]


def _logits_stats(X, W2s, cbias, blk_n):
    b, n, d = X.shape
    total = W2s.shape[2]
    grid = n // blk_n
    return pl.pallas_call(
        _k4_body,
        grid=(grid,),
        in_specs=[
            pl.BlockSpec((b, blk_n, d), lambda i: (0, i, 0)),
            pl.BlockSpec((b, d, total), lambda i: (0, 0, 0)),
            pl.BlockSpec((b, total), lambda i: (0, 0)),
        ],
        out_specs=[
            pl.BlockSpec((b, blk_n, total), lambda i: (0, i, 0)),
            pl.BlockSpec((b, total), lambda i: (0, 0)),
            pl.BlockSpec((b, total), lambda i: (0, 0)),
        ],
        out_shape=[
            jax.ShapeDtypeStruct((b, n, total), jnp.float32),
            jax.ShapeDtypeStruct((b, total), jnp.float32),
            jax.ShapeDtypeStruct((b, total), jnp.float32),
        ],
        scratch_shapes=[
            pltpu.VMEM((b, total), jnp.float32),
            pltpu.VMEM((b, total), jnp.float32),
        ],
    )(X, W2s, cbias)


def _k5_body(uniform, a_ref, m_ref, s_ref, mask_ref, out_ref):
    a = a_ref[...]
    m = m_ref[...][:, None, :]
    inv = (1.0 / s_ref[...])[:, None, :]
    val = jnp.exp(a - m) * inv
    sel = mask_ref[...][:, None, :] > 0.5
    out_ref[...] = jnp.where(sel, val, uniform)


def _finalize(A, M, S, mask, blk_n):
    b, n, total = A.shape
    grid = n // blk_n
    body = functools.partial(_k5_body, 1.0 / n)
    return pl.pallas_call(
        body,
        grid=(grid,),
        in_specs=[
            pl.BlockSpec((b, blk_n, total), lambda i: (0, i, 0)),
            pl.BlockSpec((b, total), lambda i: (0, 0)),
            pl.BlockSpec((b, total), lambda i: (0, 0)),
            pl.BlockSpec((b, total), lambda i: (0, 0)),
        ],
        out_specs=pl.BlockSpec((b, blk_n, total), lambda i: (0, i, 0)),
        out_shape=jax.ShapeDtypeStruct((b, n, total), jnp.float32),
    )(A, M, S, mask)


def kernel(X, prototype_base, Wc, bc, Wp, bp):
    b, n, d = X.shape
    total = prototype_base.shape[0]
    sum_X, max_X = _reduce_x(X, blk_n=512)
    base_plus_bc = prototype_base + bc.reshape(total, d)
    protos = _prototypes(sum_X, max_X, Wc, base_plus_bc, n, e_per_blk=2)
    W2s, cbias, gs = _edge_weights(protos, Wp, bp, sum_X, n)
    mask = _topk_mask_sc(gs)
    A, M, S = _logits_stats(X, W2s, cbias, blk_n=512)
    return _finalize(A, M, S, mask, blk_n=512)


# speedup vs baseline: 1.0628x; 1.0628x over previous
import functools
import math

import jax
import jax.numpy as jnp
from jax import lax
from jax.experimental import pallas as pl
from jax.experimental.pallas import tpu as pltpu
from jax.experimental.pallas import tpu_sc as plsc

_H = 4
_K = 16
_LANES = 16


def _s1_body(nx, nwc, e_per_blk, inv_n, scale, n_tokens,
             x_ref, wc_ref, base_ref, bc_ref, wp_ref, bp_ref,
             w2_ref, cb_ref, gs_ref,
             sum_scr, max_scr, proto_scr):
    i = pl.program_id(0)
    d = x_ref.shape[2]

    @pl.when(i < nx)
    def _():
        x = x_ref[...]
        s = jnp.sum(x, axis=1)
        m = jnp.max(x, axis=1)

        @pl.when(i == 0)
        def _():
            sum_scr[...] = s
            max_scr[...] = m

        @pl.when(i > 0)
        def _():
            sum_scr[...] = sum_scr[...] + s
            max_scr[...] = jnp.maximum(max_scr[...], m)

    @pl.when(i >= nx)
    def _():
        j = i - nx
        avg = sum_scr[...] * inv_n
        mx = max_scr[...]
        for e in range(e_per_blk):
            sl = pl.ds(e * d, d)
            off = jnp.dot(avg, wc_ref[0:d, sl],
                          preferred_element_type=jnp.float32)
            off = off + jnp.dot(mx, wc_ref[d:2 * d, sl],
                                preferred_element_type=jnp.float32)
            off = off + base_ref[0, e][None, :] + bc_ref[0, e][None, :]
            proto_scr[pl.ds(j * e_per_blk + e, 1)] = off[None]

    @pl.when(i == nx + nwc - 1)
    def _():
        b = sum_scr.shape[0]
        for bi in range(b):
            p = proto_scr[:, bi, :]
            w2 = lax.dot_general(wp_ref[...], p, (((1,), (1,)), ((), ())),
                                 preferred_element_type=jnp.float32) * scale
            w2_ref[bi] = w2
            cb = lax.dot_general(bp_ref[...], p, (((1,), (1,)), ((), ())),
                                 preferred_element_type=jnp.float32) * scale
            sx = sum_scr[pl.ds(bi, 1), :]
            gs = jnp.dot(sx, w2, preferred_element_type=jnp.float32) \
                + n_tokens * cb
            cb_ref[pl.ds(bi, 1), :] = cb
            gs_ref[pl.ds(bi, 1), :] = gs


def _edge_weights(X, Wc, base, bc, Wp, bp, blk_n, e_per_blk):
    b, n, d = X.shape
    total = base.shape[0]
    nx = n // blk_n
    nwc = total // e_per_blk
    scale = 1.0 / (_H * math.sqrt(d / _H))
    body = functools.partial(_s1_body, nx, nwc, e_per_blk, 1.0 / n, scale,
                             float(n))
    last = nx - 1

    return pl.pallas_call(
        body,
        grid=(nx + nwc,),
        in_specs=[
            pl.BlockSpec((b, blk_n, d),
                         lambda i: (0, jnp.minimum(i, last), 0)),
            pl.BlockSpec((2 * d, e_per_blk * d),
                         lambda i: (0, jnp.maximum(i - nx, 0))),
            pl.BlockSpec((1, e_per_blk, d),
                         lambda i: (jnp.maximum(i - nx, 0), 0, 0)),
            pl.BlockSpec((1, e_per_blk, d),
                         lambda i: (jnp.maximum(i - nx, 0), 0, 0)),
            pl.BlockSpec((d, d), lambda i: (0, 0)),
            pl.BlockSpec((1, d), lambda i: (0, 0)),
        ],
        out_specs=[
            pl.BlockSpec((b, d, total), lambda i: (0, 0, 0)),
            pl.BlockSpec((b, total), lambda i: (0, 0)),
            pl.BlockSpec((b, total), lambda i: (0, 0)),
        ],
        out_shape=[
            jax.ShapeDtypeStruct((b, d, total), jnp.float32),
            jax.ShapeDtypeStruct((b, total), jnp.float32),
            jax.ShapeDtypeStruct((b, total), jnp.float32),
        ],
        scratch_shapes=[
            pltpu.VMEM((b, d), jnp.float32),
            pltpu.VMEM((b, d), jnp.float32),
            pltpu.VMEM((total, b, d), jnp.float32),
        ],
    )(X, Wc, base.reshape(nwc, e_per_blk, d), bc.reshape(nwc, e_per_blk, d),
      Wp, bp.reshape(1, d))


def _sc_mask_body(batches, total, gs_hbm, out_hbm, row_v, mask_v):
    cid = lax.axis_index("c")
    sid = lax.axis_index("s")
    wid = sid * 2 + cid
    nchunk = total // _LANES

    @pl.when(wid < batches)
    def _():
        base = wid * total
        pltpu.sync_copy(gs_hbm.at[pl.ds(base, total)],
                        row_v.at[pl.ds(0, total)])
        chunks = [row_v[pl.ds(i * _LANES, _LANES)] for i in range(nchunk)]
        lane = lax.iota(jnp.int32, _LANES)
        eidx = [lane + i * _LANES for i in range(nchunk)]
        one = jnp.ones((_LANES,), jnp.int32)
        zero = jnp.zeros((_LANES,), jnp.int32)

        def body(j, accs):
            s = row_v[pl.ds(j, _LANES)][0]
            sj = jnp.full((_LANES,), s, jnp.float32)
            out = []
            for i in range(nchunk):
                gt = sj > chunks[i]
                eq = (sj == chunks[i]) & (j < eidx[i])
                out.append(accs[i] + jnp.where(gt, one, zero)
                           + jnp.where(eq, one, zero))
            return tuple(out)

        accs = lax.fori_loop(0, total, body, tuple([zero] * nchunk))
        fone = jnp.ones((_LANES,), jnp.float32)
        fzero = jnp.zeros((_LANES,), jnp.float32)
        for i in range(nchunk):
            mask_v[pl.ds(i * _LANES, _LANES)] = jnp.where(accs[i] < _K,
                                                          fone, fzero)
        pltpu.sync_copy(mask_v, out_hbm.at[pl.ds(base, total)])


def _topk_mask_sc(gs):
    b, total = gs.shape
    mesh = plsc.VectorSubcoreMesh(core_axis_name="c", subcore_axis_name="s")
    body = functools.partial(_sc_mask_body, b, total)
    kern = pl.kernel(
        body,
        out_type=jax.ShapeDtypeStruct((b * total,), jnp.float32),
        mesh=mesh,
        scratch_types=[
            pltpu.VMEM((total + _LANES,), jnp.float32),
            pltpu.VMEM((total,), jnp.float32),
        ],
    )
    return kern(gs.reshape(-1)).reshape(b, total)


def _s2_body(nx, uniform, x_ref, w2_ref, cb_ref, mask_ref, out_ref,
             m_scr, s_scr, a_scr):
    i = pl.program_id(0)
    b = x_ref.shape[0]

    @pl.when(i < nx)
    def _():
        for bi in range(b):
            xb = x_ref[bi].astype(jnp.bfloat16)
            wb = w2_ref[bi].astype(jnp.bfloat16)
            a = jnp.dot(xb, wb, preferred_element_type=jnp.float32)
            a = a + cb_ref[bi][None, :]
            a_scr[i, bi] = a
            m_blk = jnp.max(a, axis=0)

            @pl.when(i == 0)
            def _():
                m_scr[bi] = m_blk
                s_scr[bi] = jnp.sum(jnp.exp(a - m_blk[None, :]), axis=0)

            @pl.when(i > 0)
            def _():
                m_old = m_scr[bi]
                m_new = jnp.maximum(m_old, m_blk)
                s_scr[bi] = s_scr[bi] * jnp.exp(m_old - m_new) + \
                    jnp.sum(jnp.exp(a - m_new[None, :]), axis=0)
                m_scr[bi] = m_new

    @pl.when(i >= nx)
    def _():
        j = i - nx
        a = a_scr[j]
        m = m_scr[...][:, None, :]
        inv = (1.0 / s_scr[...])[:, None, :]
        sel = mask_ref[...][:, None, :] > 0.5
        out_ref[...] = jnp.where(sel, jnp.exp(a - m) * inv, uniform)


def _attn_output(X, W2s, cbias, mask, blk_n):
    b, n, d = X.shape
    total = W2s.shape[2]
    nx = n // blk_n
    body = functools.partial(_s2_body, nx, 1.0 / n)
    last = nx - 1
    return pl.pallas_call(
        body,
        grid=(2 * nx,),
        in_specs=[
            pl.BlockSpec((b, blk_n, d),
                         lambda i: (0, jnp.minimum(i, last), 0)),
            pl.BlockSpec((b, d, total), lambda i: (0, 0, 0)),
            pl.BlockSpec((b, total), lambda i: (0, 0)),
            pl.BlockSpec((b, total), lambda i: (0, 0)),
        ],
        out_specs=pl.BlockSpec((b, blk_n, total),
                               lambda i: (0, jnp.maximum(i - nx, 0), 0)),
        out_shape=jax.ShapeDtypeStruct((b, n, total), jnp.float32),
        scratch_shapes=[
            pltpu.VMEM((b, total), jnp.float32),
            pltpu.VMEM((b, total), jnp.float32),
            pltpu.VMEM((nx, b, blk_n, total), jnp.float32),
        ],
    )(X, W2s, cbias, mask)


def kernel(X, prototype_base, Wc, bc, Wp, bp):
    b, n, d = X.shape
    total = prototype_base.shape[0]
    W2s, cbias, gs = _edge_weights(X, Wc, prototype_base, bc, Wp, bp,
                                   blk_n=512, e_per_blk=2)
    mask = _topk_mask_sc(gs)
    return _attn_output(X, W2s, cbias, mask, blk_n=512)


# speedup vs baseline: 1.1276x; 1.0610x over previous
import functools
import math

import jax
import jax.numpy as jnp
from jax import lax
from jax.experimental import pallas as pl
from jax.experimental.pallas import tpu as pltpu
from jax.experimental.pallas import tpu_sc as plsc

_H = 4
_K = 16
_LANES = 16


def _s1_body(nx, nwc, e_per_blk, inv_n, scale, n_tokens,
             x_ref, wc_ref, base_ref, bc_ref, wp_ref, bp_ref,
             w2_ref, cb_ref, gs_ref,
             sum_scr, max_scr, proto_scr):
    i = pl.program_id(0)
    d = x_ref.shape[2]

    @pl.when(i < nx)
    def _():
        x = x_ref[...]
        s = jnp.sum(x, axis=1)
        m = jnp.max(x, axis=1)

        @pl.when(i == 0)
        def _():
            sum_scr[...] = s
            max_scr[...] = m

        @pl.when(i > 0)
        def _():
            sum_scr[...] = sum_scr[...] + s
            max_scr[...] = jnp.maximum(max_scr[...], m)

    @pl.when(i >= nx)
    def _():
        j = i - nx
        avg = sum_scr[...] * inv_n
        mx = max_scr[...]
        for e in range(e_per_blk):
            sl = pl.ds(e * d, d)
            off = jnp.dot(avg, wc_ref[0:d, sl],
                          preferred_element_type=jnp.float32)
            off = off + jnp.dot(mx, wc_ref[d:2 * d, sl],
                                preferred_element_type=jnp.float32)
            off = off + base_ref[0, e][None, :] + bc_ref[0, e][None, :]
            proto_scr[pl.ds(j * e_per_blk + e, 1)] = off[None]

    @pl.when(i == nx + nwc - 1)
    def _():
        b = sum_scr.shape[0]
        for bi in range(b):
            p = proto_scr[:, bi, :]
            w2 = lax.dot_general(wp_ref[...], p, (((1,), (1,)), ((), ())),
                                 preferred_element_type=jnp.float32) * scale
            w2_ref[bi] = w2
            cb = lax.dot_general(bp_ref[...], p, (((1,), (1,)), ((), ())),
                                 preferred_element_type=jnp.float32) * scale
            sx = sum_scr[pl.ds(bi, 1), :]
            gs = jnp.dot(sx, w2, preferred_element_type=jnp.float32) \
                + n_tokens * cb
            cb_ref[pl.ds(bi, 1), :] = cb
            gs_ref[pl.ds(bi, 1), :] = gs


def _edge_weights(X, Wc, base, bc, Wp, bp, blk_n, e_per_blk):
    b, n, d = X.shape
    total = base.shape[0]
    nx = n // blk_n
    nwc = total // e_per_blk
    scale = 1.0 / (_H * math.sqrt(d / _H))
    body = functools.partial(_s1_body, nx, nwc, e_per_blk, 1.0 / n, scale,
                             float(n))
    last = nx - 1

    return pl.pallas_call(
        body,
        grid=(nx + nwc,),
        in_specs=[
            pl.BlockSpec((b, blk_n, d),
                         lambda i: (0, jnp.minimum(i, last), 0)),
            pl.BlockSpec((2 * d, e_per_blk * d),
                         lambda i: (0, jnp.maximum(i - nx, 0))),
            pl.BlockSpec((1, e_per_blk, d),
                         lambda i: (jnp.maximum(i - nx, 0), 0, 0)),
            pl.BlockSpec((1, e_per_blk, d),
                         lambda i: (jnp.maximum(i - nx, 0), 0, 0)),
            pl.BlockSpec((d, d), lambda i: (0, 0)),
            pl.BlockSpec((1, d), lambda i: (0, 0)),
        ],
        out_specs=[
            pl.BlockSpec((b, d, total), lambda i: (0, 0, 0)),
            pl.BlockSpec((b, total), lambda i: (0, 0)),
            pl.BlockSpec((b, total), lambda i: (0, 0)),
        ],
        out_shape=[
            jax.ShapeDtypeStruct((b, d, total), jnp.float32),
            jax.ShapeDtypeStruct((b, total), jnp.float32),
            jax.ShapeDtypeStruct((b, total), jnp.float32),
        ],
        scratch_shapes=[
            pltpu.VMEM((b, d), jnp.float32),
            pltpu.VMEM((b, d), jnp.float32),
            pltpu.VMEM((total, b, d), jnp.float32),
        ],
    )(X, Wc, base.reshape(nwc, e_per_blk, d), bc.reshape(nwc, e_per_blk, d),
      Wp, bp.reshape(1, d))


def _sc_mask_body(batches, total, gs_hbm, out_hbm, row_v, mask_v):
    cid = lax.axis_index("c")
    sid = lax.axis_index("s")
    wid = sid * 2 + cid
    nchunk = total // _LANES

    @pl.when(wid < batches)
    def _():
        base = wid * total
        pltpu.sync_copy(gs_hbm.at[pl.ds(base, total)],
                        row_v.at[pl.ds(0, total)])
        chunks = [row_v[pl.ds(i * _LANES, _LANES)] for i in range(nchunk)]
        lane = lax.iota(jnp.int32, _LANES)
        eidx = [lane + i * _LANES for i in range(nchunk)]
        one = jnp.ones((_LANES,), jnp.int32)
        zero = jnp.zeros((_LANES,), jnp.int32)

        def body(j, accs):
            s = row_v[pl.ds(j, _LANES)][0]
            sj = jnp.full((_LANES,), s, jnp.float32)
            out = []
            for i in range(nchunk):
                gt = sj > chunks[i]
                eq = (sj == chunks[i]) & (j < eidx[i])
                out.append(accs[i] + jnp.where(gt, one, zero)
                           + jnp.where(eq, one, zero))
            return tuple(out)

        accs = lax.fori_loop(0, total, body, tuple([zero] * nchunk))
        fone = jnp.ones((_LANES,), jnp.float32)
        fzero = jnp.zeros((_LANES,), jnp.float32)
        for i in range(nchunk):
            mask_v[pl.ds(i * _LANES, _LANES)] = jnp.where(accs[i] < _K,
                                                          fone, fzero)
        pltpu.sync_copy(mask_v, out_hbm.at[pl.ds(base, total)])


def _topk_mask_sc(gs):
    b, total = gs.shape
    mesh = plsc.VectorSubcoreMesh(core_axis_name="c", subcore_axis_name="s")
    body = functools.partial(_sc_mask_body, b, total)
    kern = pl.kernel(
        body,
        out_type=jax.ShapeDtypeStruct((b * total,), jnp.float32),
        mesh=mesh,
        scratch_types=[
            pltpu.VMEM((total + _LANES,), jnp.float32),
            pltpu.VMEM((total,), jnp.float32),
        ],
    )
    return kern(gs.reshape(-1)).reshape(b, total)


def _s2_body(nx, uniform, x_ref, w2_ref, cb_ref, mask_ref, out_ref,
             m_scr, s_scr, mh_scr, a_scr):
    i = pl.program_id(0)
    b = x_ref.shape[0]

    @pl.when(i < nx)
    def _():
        def dot_b(bi):
            xb = x_ref[bi].astype(jnp.bfloat16)
            wb = w2_ref[bi].astype(jnp.bfloat16)
            a = jnp.dot(xb, wb, preferred_element_type=jnp.float32)
            return a + cb_ref[bi][None, :]

        def stats_b(bi, a):
            m_blk = jnp.max(a, axis=0)

            @pl.when(i == 0)
            def _():
                e = jnp.exp(a - m_blk[None, :])
                a_scr[i, bi] = e
                m_scr[bi] = m_blk
                mh_scr[i, bi] = m_blk
                s_scr[bi] = jnp.sum(e, axis=0)

            @pl.when(i > 0)
            def _():
                m_old = m_scr[bi]
                m_new = jnp.maximum(m_old, m_blk)
                e = jnp.exp(a - m_new[None, :])
                a_scr[i, bi] = e
                mh_scr[i, bi] = m_new
                s_scr[bi] = s_scr[bi] * jnp.exp(m_old - m_new) + \
                    jnp.sum(e, axis=0)
                m_scr[bi] = m_new

        a_prev = dot_b(0)
        for bi in range(1, b):
            a_cur = dot_b(bi)
            stats_b(bi - 1, a_prev)
            a_prev = a_cur
        stats_b(b - 1, a_prev)

    @pl.when(i >= nx)
    def _():
        j = i - nx
        e = a_scr[j]
        fac = jnp.exp(mh_scr[j] - m_scr[...]) / s_scr[...]
        sel = mask_ref[...][:, None, :] > 0.5
        out_ref[...] = jnp.where(sel, e * fac[:, None, :], uniform)


def _attn_output(X, W2s, cbias, mask, blk_n):
    b, n, d = X.shape
    total = W2s.shape[2]
    nx = n // blk_n
    body = functools.partial(_s2_body, nx, 1.0 / n)
    last = nx - 1
    return pl.pallas_call(
        body,
        grid=(2 * nx,),
        in_specs=[
            pl.BlockSpec((b, blk_n, d),
                         lambda i: (0, jnp.minimum(i, last), 0)),
            pl.BlockSpec((b, d, total), lambda i: (0, 0, 0)),
            pl.BlockSpec((b, total), lambda i: (0, 0)),
            pl.BlockSpec((b, total), lambda i: (0, 0)),
        ],
        out_specs=pl.BlockSpec((b, blk_n, total),
                               lambda i: (0, jnp.maximum(i - nx, 0), 0)),
        out_shape=jax.ShapeDtypeStruct((b, n, total), jnp.float32),
        scratch_shapes=[
            pltpu.VMEM((b, total), jnp.float32),
            pltpu.VMEM((b, total), jnp.float32),
            pltpu.VMEM((nx, b, total), jnp.float32),
            pltpu.VMEM((nx, b, blk_n, total), jnp.float32),
        ],
    )(X, W2s, cbias, mask)


def kernel(X, prototype_base, Wc, bc, Wp, bp):
    b, n, d = X.shape
    total = prototype_base.shape[0]
    W2s, cbias, gs = _edge_weights(X, Wc, prototype_base, bc, Wp, bp,
                                   blk_n=512, e_per_blk=2)
    mask = _topk_mask_sc(gs)
    return _attn_output(X, W2s, cbias, mask, blk_n=1024)
    return _attn_output(X, W2s, cbias, mask, blk_n=512)
